# baseline (speedup 1.0000x reference)
import functools

import jax
import jax.numpy as jnp
from jax import lax
from jax.experimental import pallas as pl
from jax.experimental.pallas import tpu as pltpu
from jax.experimental.pallas import tpu_sc as plsc

N_NODES = 10000
N_EDGES = 320000
D_FEAT = 128
NUM_LAYERS = 3
NUM_OUTPUTS = 1000

NC = 2
NS = 16
NW = NC * NS
CHUNK = 80
N_CHUNKS = N_EDGES // CHUNK
CPT = N_CHUNKS // NW
NPT = 624
TAIL = N_NODES - NPT * NS
BCHUNK = 80
NB_CHUNKS = N_NODES // BCHUNK
FB = D_FEAT // 16


def _shuffle_xor(v, shift):
    idx = jnp.reshape(
        jax.lax.iota(jnp.int32, 16) ^ jnp.int32(shift), (16, 1))
    return lax.gather(
        v, idx,
        dimension_numbers=lax.GatherDimensionNumbers(
            offset_dims=(), collapsed_slice_dims=(0,), start_index_map=(0,)),
        slice_sizes=(1,), mode=lax.GatherScatterMode.PROMISE_IN_BOUNDS)


def _mesh():
    return plsc.VectorSubcoreMesh(
        core_axis_name="c", subcore_axis_name="s", num_cores=NC,
        num_subcores=NS)


def _lane_bcast(wblk, l):
    idx = jnp.full((16, 1), l, jnp.int32)
    return lax.gather(
        wblk, idx,
        dimension_numbers=lax.GatherDimensionNumbers(
            offset_dims=(), collapsed_slice_dims=(0,), start_index_map=(0,)),
        slice_sizes=(1,), mode=lax.GatherScatterMode.PROMISE_IN_BOUNDS)


@functools.partial(
    pl.kernel,
    out_type=jax.ShapeDtypeStruct((NC, N_NODES, D_FEAT), jnp.float32),
    mesh=_mesh(),
    scratch_types=[
        pltpu.VMEM_SHARED((N_NODES, D_FEAT), jnp.float32),
    ] + [pltpu.VMEM((CHUNK, D_FEAT), jnp.float32) for _ in range(4)]
      + [pltpu.VMEM((CHUNK,), jnp.int32) for _ in range(4)]
      + [pltpu.VMEM((CHUNK,), jnp.int32) for _ in range(4)]
      + [pltpu.VMEM((CHUNK,), jnp.float32) for _ in range(4)]
      + [pltpu.SemaphoreType.DMA,
         pltpu.SemaphoreType.DMA,
         pltpu.SemaphoreType.DMA],
)
def _phase_a(vals_hbm, src_hbm, dst_hbm, w_hbm, zeros_hbm, part_hbm,
             acc_sp, r0, r1, r2, r3, s0, s1, s2, s3, d0, d1, d2, d3,
             w0, w1, w2, w3, semI, semG, semS):
    rows = [r0, r1, r2, r3]
    srcs = [s0, s1, s2, s3]
    dsts = [d0, d1, d2, d3]
    ws = [w0, w1, w2, w3]
    cid = lax.axis_index("c")
    sid = lax.axis_index("s")
    wid = sid * NC + cid

    pltpu.sync_copy(zeros_hbm.at[pl.ds(0, NPT)],
                    acc_sp.at[pl.ds(sid * NPT, NPT)])

    @pl.when(sid == NS - 1)
    def _zero_tail():
        pltpu.sync_copy(zeros_hbm.at[pl.ds(0, TAIL)],
                        acc_sp.at[pl.ds(NPT * NS, TAIL)])

    def coff(k):
        return (wid + NW * k) * CHUNK

    def stage_idx(k, b):
        pltpu.async_copy(src_hbm.at[pl.ds(coff(k), CHUNK)], srcs[b], semI)
        pltpu.async_copy(dst_hbm.at[pl.ds(coff(k), CHUNK)], dsts[b], semI)
        pltpu.async_copy(w_hbm.at[pl.ds(coff(k), CHUNK)], ws[b], semI)

    def wait_idx(k, b):
        pltpu.make_async_copy(src_hbm.at[pl.ds(coff(k), CHUNK)], srcs[b],
                              semI).wait()
        pltpu.make_async_copy(dst_hbm.at[pl.ds(coff(k), CHUNK)], dsts[b],
                              semI).wait()
        pltpu.make_async_copy(w_hbm.at[pl.ds(coff(k), CHUNK)], ws[b],
                              semI).wait()

    def start_gather(b):
        pltpu.async_copy(vals_hbm.at[srcs[b]], rows[b], semG)

    def wait_gather(b):
        pltpu.make_async_copy(vals_hbm.at[srcs[b]], rows[b], semG).wait()

    def start_scatter(b):
        pltpu.async_copy(rows[b], acc_sp.at[dsts[b]], semS, add=True)

    def wait_scatter(b):
        pltpu.make_async_copy(rows[b], acc_sp.at[dsts[b]], semS).wait()

    stage_idx(0, 0)
    wait_idx(0, 0)
    start_gather(0)
    stage_idx(1, 1)

    plsc.subcore_barrier()

    def chunk_body(k, carry):
        b4 = lax.rem(k, 4)

        def run(b):
            nb = (b + 1) % 4
            fb = (b + 2) % 4
            wait_gather(b)

            @pl.when(k + 1 < CPT)
            def _next_gather():
                wait_idx(k + 1, nb)
                start_gather(nb)

            def grp_body(g, c):
                base = g * 16
                wblk = ws[b][pl.ds(base, 16)]
                for l in range(16):
                    e = base + l
                    wv = _lane_bcast(wblk, l)
                    for j in range(FB):
                        rows[b][e, pl.ds(j * 16, 16)] = (
                            rows[b][e, pl.ds(j * 16, 16)] * wv)
                return c

            lax.fori_loop(0, CHUNK // 16, grp_body, 0)

            @pl.when(k > 1)
            def _drain_scatter_k2():
                wait_scatter(fb)

            start_scatter(b)

            @pl.when(k + 2 < CPT)
            def _next_idx():
                stage_idx(k + 2, fb)

        for bb in range(4):
            @pl.when(b4 == bb)
            def _run(bb=bb):
                run(bb)

        return carry

    lax.fori_loop(0, CPT, chunk_body, 0)
    wait_scatter((CPT - 2) % 4)
    wait_scatter((CPT - 1) % 4)
    plsc.subcore_barrier()
    pltpu.sync_copy(acc_sp.at[pl.ds(sid * NPT, NPT)],
                    part_hbm.at[cid, pl.ds(sid * NPT, NPT)])

    @pl.when(sid == NS - 1)
    def _dump_tail():
        pltpu.sync_copy(acc_sp.at[pl.ds(NPT * NS, TAIL)],
                        part_hbm.at[cid, pl.ds(NPT * NS, TAIL)])


@functools.partial(
    pl.kernel,
    out_type=jax.ShapeDtypeStruct((N_NODES, D_FEAT), jnp.float32),
    mesh=_mesh(),
    scratch_types=[
        pltpu.VMEM((2, BCHUNK, D_FEAT), jnp.float32),
        pltpu.VMEM((2, BCHUNK, D_FEAT), jnp.float32),
        pltpu.VMEM((2, BCHUNK, D_FEAT), jnp.float32),
        pltpu.SemaphoreType.DMA,
        pltpu.SemaphoreType.DMA,
    ],
)
def _phase_b(part_hbm, out_hbm, a_v, b_v, o_v, semL, semO):
    cid = lax.axis_index("c")
    sid = lax.axis_index("s")
    wid = sid * NC + cid

    rem = NB_CHUNKS - (NB_CHUNKS // NW) * NW
    cnt = jnp.where(wid < rem, NB_CHUNKS // NW + 1, NB_CHUNKS // NW)

    def boff(k):
        return (wid + NW * k) * BCHUNK

    def stage(k, b):
        pltpu.async_copy(part_hbm.at[0, pl.ds(boff(k), BCHUNK)], a_v.at[b],
                         semL)
        pltpu.async_copy(part_hbm.at[1, pl.ds(boff(k), BCHUNK)], b_v.at[b],
                         semL)

    def wait_stage(k, b):
        pltpu.make_async_copy(part_hbm.at[0, pl.ds(boff(k), BCHUNK)],
                              a_v.at[b], semL).wait()
        pltpu.make_async_copy(part_hbm.at[1, pl.ds(boff(k), BCHUNK)],
                              b_v.at[b], semL).wait()

    def start_store(k, b):
        pltpu.async_copy(o_v.at[b], out_hbm.at[pl.ds(boff(k), BCHUNK)],
                         semO)

    def wait_store(k, b):
        pltpu.make_async_copy(o_v.at[b], out_hbm.at[pl.ds(boff(k), BCHUNK)],
                              semO).wait()

    stage(0, 0)

    @pl.when(cnt > 1)
    def _prefetch1():
        stage(1, 1)

    def chunk_body(k, carry):
        b2 = lax.rem(k, 2)

        def run(b):
            @pl.when(k > 1)
            def _drain_store():
                wait_store(k - 2, b)

            wait_stage(k, b)

            def node_body(i, c):
                vs = [a_v[b, i, pl.ds(j * 16, 16)] +
                      b_v[b, i, pl.ds(j * 16, 16)] for j in range(FB)]
                m = vs[0]
                for j in range(1, FB):
                    m = jnp.maximum(m, vs[j])
                for sh in (8, 4, 2, 1):
                    m = jnp.maximum(m, _shuffle_xor(m, sh))
                es = [jnp.exp(v - m) for v in vs]
                s = es[0]
                for j in range(1, FB):
                    s = s + es[j]
                for sh in (8, 4, 2, 1):
                    s = s + _shuffle_xor(s, sh)
                r = 1.0 / s
                for j in range(FB):
                    o_v[b, i, pl.ds(j * 16, 16)] = es[j] * r
                return c

            lax.fori_loop(0, BCHUNK, node_body, 0)
            start_store(k, b)

            @pl.when(k + 2 < cnt)
            def _next_stage():
                stage(k + 2, b)

        @pl.when(b2 == 0)
        def _b0():
            run(0)

        @pl.when(b2 == 1)
        def _b1():
            run(1)

        return carry

    lax.fori_loop(0, cnt, chunk_body, 0)

    lb1 = lax.rem(cnt - 1, 2)
    for bb in range(2):
        @pl.when(lb1 == bb)
        def _d1(bb=bb):
            wait_store(cnt - 1, bb)

        @pl.when((cnt > 1) & (lax.rem(cnt - 2, 2) == bb))
        def _d2(bb=bb):
            wait_store(cnt - 2, bb)


def kernel(x, edge_index, edge_weight):
    src = edge_index[0].astype(jnp.int32)
    dst = edge_index[1].astype(jnp.int32)
    w = edge_weight.astype(jnp.float32)
    zeros = jnp.zeros((NPT, D_FEAT), jnp.float32)
    vals = x
    for _ in range(NUM_LAYERS):
        part = _phase_a(vals, src, dst, w, zeros)
        vals = _phase_b(part)
    return vals[N_NODES - NUM_OUTPUTS:]


# speedup vs baseline: 1.1915x; 1.1915x over previous
import functools

import jax
import jax.numpy as jnp
from jax import lax
from jax.experimental import pallas as pl
from jax.experimental.pallas import tpu as pltpu
from jax.experimental.pallas import tpu_sc as plsc

N_NODES = 10000
N_EDGES = 320000
D_FEAT = 128
NUM_LAYERS = 3
NUM_OUTPUTS = 1000

NC = 2
NS = 16
NW = NC * NS
CHUNK = 128
N_CHUNKS = N_EDGES // CHUNK
NPT = 624
TAIL = N_NODES - NPT * NS
BCHUNK = 80
NB_CHUNKS = N_NODES // BCHUNK
FB = D_FEAT // 16


def _shuffle_xor(v, shift):
    idx = jnp.reshape(
        jax.lax.iota(jnp.int32, 16) ^ jnp.int32(shift), (16, 1))
    return lax.gather(
        v, idx,
        dimension_numbers=lax.GatherDimensionNumbers(
            offset_dims=(), collapsed_slice_dims=(0,), start_index_map=(0,)),
        slice_sizes=(1,), mode=lax.GatherScatterMode.PROMISE_IN_BOUNDS)


def _mesh():
    return plsc.VectorSubcoreMesh(
        core_axis_name="c", subcore_axis_name="s", num_cores=NC,
        num_subcores=NS)


def _lane_bcast(wblk, l):
    idx = jnp.full((16, 1), l, jnp.int32)
    return lax.gather(
        wblk, idx,
        dimension_numbers=lax.GatherDimensionNumbers(
            offset_dims=(), collapsed_slice_dims=(0,), start_index_map=(0,)),
        slice_sizes=(1,), mode=lax.GatherScatterMode.PROMISE_IN_BOUNDS)


@functools.partial(
    pl.kernel,
    out_type=jax.ShapeDtypeStruct((NC, N_NODES, D_FEAT), jnp.float32),
    mesh=_mesh(),
    scratch_types=[
        pltpu.VMEM_SHARED((N_NODES, D_FEAT), jnp.float32),
        pltpu.VMEM((3, CHUNK, D_FEAT), jnp.float32),
        pltpu.VMEM((3, 2, CHUNK), jnp.int32),
        pltpu.VMEM((3, CHUNK), jnp.float32),
        pltpu.SemaphoreType.DMA,
        pltpu.SemaphoreType.DMA,
        pltpu.SemaphoreType.DMA,
    ],
)
def _phase_a(vals_hbm, edges_hbm, w_hbm, zeros_hbm, part_hbm,
             acc_sp, rows_v, e_v, w_v, semI, semG, semS):
    cid = lax.axis_index("c")
    sid = lax.axis_index("s")
    wid = sid * NC + cid

    pltpu.sync_copy(zeros_hbm.at[pl.ds(0, NPT)],
                    acc_sp.at[pl.ds(sid * NPT, NPT)])

    @pl.when(sid == NS - 1)
    def _zero_tail():
        pltpu.sync_copy(zeros_hbm.at[pl.ds(0, TAIL)],
                        acc_sp.at[pl.ds(NPT * NS, TAIL)])

    rem = N_CHUNKS - (N_CHUNKS // NW) * NW
    cnt = jnp.where(wid < rem, N_CHUNKS // NW + 1, N_CHUNKS // NW)

    def coff(k):
        return (wid + NW * k) * CHUNK

    def stage_idx(k, b):
        pltpu.async_copy(edges_hbm.at[:, pl.ds(coff(k), CHUNK)],
                         e_v.at[b], semI)
        pltpu.async_copy(w_hbm.at[pl.ds(coff(k), CHUNK)], w_v.at[b], semI)

    def wait_idx(k, b):
        pltpu.make_async_copy(edges_hbm.at[:, pl.ds(coff(k), CHUNK)],
                              e_v.at[b], semI).wait()
        pltpu.make_async_copy(w_hbm.at[pl.ds(coff(k), CHUNK)],
                              w_v.at[b], semI).wait()

    def start_gather(b):
        pltpu.async_copy(vals_hbm.at[e_v.at[b, 0]], rows_v.at[b], semG)

    def wait_gather(b):
        pltpu.make_async_copy(vals_hbm.at[e_v.at[b, 0]], rows_v.at[b],
                              semG).wait()

    def start_scatter(b):
        pltpu.async_copy(rows_v.at[b], acc_sp.at[e_v.at[b, 1]], semS,
                         add=True)

    def wait_scatter(b):
        pltpu.make_async_copy(rows_v.at[b], acc_sp.at[e_v.at[b, 1]],
                              semS).wait()

    stage_idx(0, 0)
    wait_idx(0, 0)
    start_gather(0)

    @pl.when(cnt > 1)
    def _prefetch1():
        stage_idx(1, 1)

    plsc.subcore_barrier()

    def chunk_body(k, carry):
        b3 = lax.rem(k, 3)

        def run(b):
            nb = (b + 1) % 3
            pb = (b + 2) % 3
            wait_gather(b)

            @pl.when(k + 1 < cnt)
            def _next_gather():
                wait_idx(k + 1, nb)
                start_gather(nb)

            def grp_body(g, c):
                base = g * 16
                wblk = w_v[b, pl.ds(base, 16)]
                for l in range(16):
                    e = base + l
                    wv = _lane_bcast(wblk, l)
                    for j in range(FB):
                        rows_v[b, e, pl.ds(j * 16, 16)] = (
                            rows_v[b, e, pl.ds(j * 16, 16)] * wv)
                return c

            lax.fori_loop(0, CHUNK // 16, grp_body, 0)

            @pl.when(k > 0)
            def _drain_prev_scatter():
                wait_scatter(pb)

            start_scatter(b)

            @pl.when(k + 2 < cnt)
            def _next_idx():
                stage_idx(k + 2, pb)

        for bb in range(3):
            @pl.when(b3 == bb)
            def _run(bb=bb):
                run(bb)

        return carry

    lax.fori_loop(0, cnt, chunk_body, 0)
    lb = lax.rem(cnt - 1, 3)
    for b in range(3):
        @pl.when(lb == b)
        def _drain_last(b=b):
            wait_scatter(b)

    plsc.subcore_barrier()
    pltpu.sync_copy(acc_sp.at[pl.ds(sid * NPT, NPT)],
                    part_hbm.at[cid, pl.ds(sid * NPT, NPT)])

    @pl.when(sid == NS - 1)
    def _dump_tail():
        pltpu.sync_copy(acc_sp.at[pl.ds(NPT * NS, TAIL)],
                        part_hbm.at[cid, pl.ds(NPT * NS, TAIL)])


def _make_phase_b(first_chunk, n_chunks):
    @functools.partial(
        pl.kernel,
        out_type=jax.ShapeDtypeStruct((N_NODES, D_FEAT), jnp.float32),
        mesh=_mesh(),
        scratch_types=[
            pltpu.VMEM((2, BCHUNK, D_FEAT), jnp.float32),
            pltpu.VMEM((2, BCHUNK, D_FEAT), jnp.float32),
            pltpu.VMEM((2, BCHUNK, D_FEAT), jnp.float32),
            pltpu.SemaphoreType.DMA,
            pltpu.SemaphoreType.DMA,
        ],
    )
    def _phase_b(part_hbm, out_hbm, a_v, b_v, o_v, semL, semO):
        cid = lax.axis_index("c")
        sid = lax.axis_index("s")
        wid = sid * NC + cid

        rem = n_chunks - (n_chunks // NW) * NW
        cnt = jnp.where(wid < rem, n_chunks // NW + 1, n_chunks // NW)

        def boff(k):
            return (first_chunk + wid + NW * k) * BCHUNK

        def stage(k, b):
            pltpu.async_copy(part_hbm.at[0, pl.ds(boff(k), BCHUNK)],
                             a_v.at[b], semL)
            pltpu.async_copy(part_hbm.at[1, pl.ds(boff(k), BCHUNK)],
                             b_v.at[b], semL)

        def wait_stage(k, b):
            pltpu.make_async_copy(part_hbm.at[0, pl.ds(boff(k), BCHUNK)],
                                  a_v.at[b], semL).wait()
            pltpu.make_async_copy(part_hbm.at[1, pl.ds(boff(k), BCHUNK)],
                                  b_v.at[b], semL).wait()

        def start_store(k, b):
            pltpu.async_copy(o_v.at[b], out_hbm.at[pl.ds(boff(k), BCHUNK)],
                             semO)

        def wait_store(k, b):
            pltpu.make_async_copy(o_v.at[b],
                                  out_hbm.at[pl.ds(boff(k), BCHUNK)],
                                  semO).wait()

        @pl.when(cnt > 0)
        def _prefetch0():
            stage(0, 0)

        @pl.when(cnt > 1)
        def _prefetch1():
            stage(1, 1)

        def chunk_body(k, carry):
            b2 = lax.rem(k, 2)

            def run(b):
                @pl.when(k > 1)
                def _drain_store():
                    wait_store(k - 2, b)

                wait_stage(k, b)

                def node_body(i, c):
                    vs = [a_v[b, i, pl.ds(j * 16, 16)] +
                          b_v[b, i, pl.ds(j * 16, 16)] for j in range(FB)]
                    m = vs[0]
                    for j in range(1, FB):
                        m = jnp.maximum(m, vs[j])
                    for sh in (8, 4, 2, 1):
                        m = jnp.maximum(m, _shuffle_xor(m, sh))
                    es = [jnp.exp(v - m) for v in vs]
                    s = es[0]
                    for j in range(1, FB):
                        s = s + es[j]
                    for sh in (8, 4, 2, 1):
                        s = s + _shuffle_xor(s, sh)
                    r = 1.0 / s
                    for j in range(FB):
                        o_v[b, i, pl.ds(j * 16, 16)] = es[j] * r
                    return c

                lax.fori_loop(0, BCHUNK, node_body, 0)
                start_store(k, b)

                @pl.when(k + 2 < cnt)
                def _next_stage():
                    stage(k + 2, b)

            @pl.when(b2 == 0)
            def _b0():
                run(0)

            @pl.when(b2 == 1)
            def _b1():
                run(1)

            return carry

        lax.fori_loop(0, cnt, chunk_body, 0)

        for bb in range(2):
            @pl.when((cnt > 0) & (lax.rem(cnt - 1, 2) == bb))
            def _d1(bb=bb):
                wait_store(cnt - 1, bb)

            @pl.when((cnt > 1) & (lax.rem(cnt - 2, 2) == bb))
            def _d2(bb=bb):
                wait_store(cnt - 2, bb)

    return _phase_b


_phase_b_full = _make_phase_b(0, NB_CHUNKS)
_OUT_FIRST_CHUNK = (N_NODES - NUM_OUTPUTS) // BCHUNK
_phase_b_last = _make_phase_b(_OUT_FIRST_CHUNK,
                              NB_CHUNKS - _OUT_FIRST_CHUNK)


def kernel(x, edge_index, edge_weight):
    src = edge_index[0].astype(jnp.int32)
    dst = edge_index[1].astype(jnp.int32)
    w = edge_weight.astype(jnp.float32)
    edges = jnp.stack([src, dst])
    zeros = jnp.zeros((NPT, D_FEAT), jnp.float32)
    vals = x
    for layer in range(NUM_LAYERS):
        part = _phase_a(vals, edges, w, zeros)
        if layer < NUM_LAYERS - 1:
            vals = _phase_b_full(part)
        else:
            vals = _phase_b_last(part)
    return vals[N_NODES - NUM_OUTPUTS:]
